# baseline (speedup 1.0000x reference)
import jax
import jax.numpy as jnp
from jax import lax
from jax.experimental import pallas as pl
from jax.experimental.pallas import tpu as pltpu
from jax.experimental.pallas import tpu_sc as plsc

BATCH = 1024
HIST = 20
VOCAB = 100000

NUM_CORES = 1
NUM_SUBCORES = 16
NUM_TILES = NUM_CORES * NUM_SUBCORES
ROWS_PER_TILE = BATCH // NUM_TILES
GROUPS = ROWS_PER_TILE // 16
WORDS_PER_TILE = HIST * ROWS_PER_TILE
CHUNK = 128
CHUNKS = WORDS_PER_TILE // CHUNK


def _sc_bow_kernel(x_hbm, w_hbm, out_hbm, idx_v, vals_v, out_v, sem):
    c = lax.axis_index("c")
    s = lax.axis_index("s")
    wid = s * NUM_CORES + c

    pltpu.sync_copy(x_hbm.at[wid], idx_v)

    w_row = w_hbm.at[0]
    copies = [
        pltpu.async_copy(w_row.at[idx_v.at[p]], vals_v.at[p], sem)
        for p in range(CHUNKS)
    ]
    for cp in copies:
        cp.wait()

    bias = jnp.zeros((16,), jnp.float32)

    def group_body(g, carry):
        def at(l):
            o = l * ROWS_PER_TILE
            return o // CHUNK, pl.ds(o % CHUNK + g * 16, 16)

        idx = []
        for l in range(HIST):
            p, sl = at(l)
            idx.append(idx_v[p, sl])
        acc = bias
        for l in range(HIST):
            dup = None
            for j in range(l):
                eq = idx[j] == idx[l]
                dup = eq if dup is None else (dup | eq)
            p, sl = at(l)
            val = vals_v[p, sl]
            if dup is not None:
                val = jnp.where(dup, 0.0, val)
            acc = acc + val
        out_v[pl.ds(g * 16, 16)] = acc
        return carry

    lax.fori_loop(0, GROUPS, group_body, 0)

    pltpu.sync_copy(out_v, out_hbm.at[pl.ds(wid * ROWS_PER_TILE,
                                            ROWS_PER_TILE)])


@jax.jit
def _bow_forward(x, W, b):
    xh = (x.reshape(NUM_TILES, ROWS_PER_TILE, HIST)
          .transpose(0, 2, 1).reshape(NUM_TILES, CHUNKS, CHUNK))

    mesh = plsc.VectorSubcoreMesh(core_axis_name="c", subcore_axis_name="s",
                                  num_cores=NUM_CORES)
    run = pl.kernel(
        _sc_bow_kernel,
        mesh=mesh,
        out_type=jax.ShapeDtypeStruct((BATCH,), jnp.float32),
        scratch_types=[
            pltpu.VMEM((CHUNKS, CHUNK), jnp.int32),
            pltpu.VMEM((CHUNKS, CHUNK), jnp.float32),
            pltpu.VMEM((ROWS_PER_TILE,), jnp.float32),
            pltpu.SemaphoreType.DMA,
        ],
    )
    out_flat = run(xh, W)
    return out_flat.reshape(BATCH, 1) + b


def kernel(x, W, b):
    return _bow_forward(x, W, b)


# speedup vs baseline: 1.0869x; 1.0869x over previous
import jax
import jax.numpy as jnp
from jax import lax
from jax.experimental import pallas as pl
from jax.experimental.pallas import tpu as pltpu
from jax.experimental.pallas import tpu_sc as plsc

BATCH = 1024
HIST = 20
VOCAB = 100000

NUM_CORES = 2
NUM_SUBCORES = 16
NUM_TILES = NUM_CORES * NUM_SUBCORES
ROWS_PER_TILE = BATCH // NUM_TILES
GROUPS = ROWS_PER_TILE // 16
WORDS_PER_TILE = HIST * ROWS_PER_TILE
CHUNK = 128
CHUNKS = WORDS_PER_TILE // CHUNK


def _sc_bow_kernel(x_hbm, w_hbm, out_hbm, idx_v, vals_v, out_v, sem):
    c = lax.axis_index("c")
    s = lax.axis_index("s")
    wid = s * NUM_CORES + c

    pltpu.sync_copy(x_hbm.at[wid], idx_v)

    w_row = w_hbm.at[0]
    copies = [
        pltpu.async_copy(w_row.at[idx_v.at[p]], vals_v.at[p], sem)
        for p in range(CHUNKS)
    ]
    for cp in copies:
        cp.wait()

    bias = jnp.zeros((16,), jnp.float32)

    def group_body(g, carry):
        def at(l):
            o = l * ROWS_PER_TILE
            return o // CHUNK, pl.ds(o % CHUNK + g * 16, 16)

        idx = []
        for l in range(HIST):
            p, sl = at(l)
            idx.append(idx_v[p, sl])
        acc = bias
        for l in range(HIST):
            dup = None
            for j in range(l):
                eq = idx[j] == idx[l]
                dup = eq if dup is None else (dup | eq)
            p, sl = at(l)
            val = vals_v[p, sl]
            if dup is not None:
                val = jnp.where(dup, 0.0, val)
            acc = acc + val
        out_v[pl.ds(g * 16, 16)] = acc
        return carry

    lax.fori_loop(0, GROUPS, group_body, 0)

    pltpu.sync_copy(out_v, out_hbm.at[pl.ds(wid * ROWS_PER_TILE,
                                            ROWS_PER_TILE)])


@jax.jit
def _bow_forward(x, W, b):
    xh = (x.reshape(NUM_TILES, ROWS_PER_TILE, HIST)
          .transpose(0, 2, 1).reshape(NUM_TILES, CHUNKS, CHUNK))

    mesh = plsc.VectorSubcoreMesh(core_axis_name="c", subcore_axis_name="s",
                                  num_cores=NUM_CORES)
    run = pl.kernel(
        _sc_bow_kernel,
        mesh=mesh,
        out_type=jax.ShapeDtypeStruct((BATCH,), jnp.float32),
        scratch_types=[
            pltpu.VMEM((CHUNKS, CHUNK), jnp.int32),
            pltpu.VMEM((CHUNKS, CHUNK), jnp.float32),
            pltpu.VMEM((ROWS_PER_TILE,), jnp.float32),
            pltpu.SemaphoreType.DMA,
        ],
    )
    out_flat = run(xh, W)
    return out_flat.reshape(BATCH, 1) + b


def kernel(x, W, b):
    return _bow_forward(x, W, b)
